# baseline (speedup 1.0000x reference)
import functools

import jax
import jax.numpy as jnp
from jax.experimental import pallas as pl

N = 10000
E = 160000
D = 128
R = 8
BN = 1000
NB = N // BN


def _dense_body(x_ref, w_ref, xw_ref):
    xw_ref[...] = jnp.dot(x_ref[...], w_ref[0],
                          preferred_element_type=jnp.float32)


def _scal_body(x_ref, w_ref, q_ref, k_ref, gq_ref, gk_ref):
    wq = jnp.einsum('rdf,fo->rd', w_ref[...], q_ref[...],
                    preferred_element_type=jnp.float32)
    wk = jnp.einsum('rdf,fo->rd', w_ref[...], k_ref[...],
                    preferred_element_type=jnp.float32)
    xb = x_ref[...]
    gq_ref[...] = jnp.dot(xb, wq.T, preferred_element_type=jnp.float32)
    gk_ref[...] = jnp.dot(xb, wk.T, preferred_element_type=jnp.float32)


@jax.jit
def _dense(x, weight, q, k_att):
    xw = pl.pallas_call(
        _dense_body,
        grid=(NB, R),
        in_specs=[
            pl.BlockSpec((BN, D), lambda i, r: (i, 0)),
            pl.BlockSpec((1, D, D), lambda i, r: (r, 0, 0)),
        ],
        out_specs=pl.BlockSpec((BN, D), lambda i, r: (r * NB + i, 0)),
        out_shape=jax.ShapeDtypeStruct((R * N, D), jnp.float32),
    )(x, weight)
    gq, gk = pl.pallas_call(
        _scal_body,
        grid=(NB,),
        in_specs=[
            pl.BlockSpec((BN, D), lambda i: (i, 0)),
            pl.BlockSpec((R, D, D), lambda i: (0, 0, 0)),
            pl.BlockSpec((D, 1), lambda i: (0, 0)),
            pl.BlockSpec((D, 1), lambda i: (0, 0)),
        ],
        out_specs=[
            pl.BlockSpec((BN, R), lambda i: (i, 0)),
            pl.BlockSpec((BN, R), lambda i: (i, 0)),
        ],
        out_shape=[
            jax.ShapeDtypeStruct((N, R), jnp.float32),
            jax.ShapeDtypeStruct((N, R), jnp.float32),
        ],
    )(x, weight, q, k_att)
    return xw, gq, gk


def kernel(x, edge_index, edge_type, weight, q, k_att, bias):
    src = edge_index[0]
    dst = edge_index[1]
    et = edge_type

    xw_flat, gq, gk = _dense(x, weight, q, k_att)

    qi = gq[dst, et]
    kj = gk[src, et]
    pre = qi + kj
    alpha = jnp.where(pre >= 0, pre, 0.2 * pre)
    ex = jnp.exp(alpha)
    denom = jax.ops.segment_sum(ex, dst, num_segments=N)
    att = ex / (denom[dst] + 1e-16)
    rows = xw_flat[et * N + src]
    aggr = jax.ops.segment_sum(att[:, None] * rows, dst, num_segments=N)
    out = aggr + bias + x
    return out, edge_index, att[:, None]


# speedup vs baseline: 10.3201x; 10.3201x over previous
import functools

import jax
import jax.numpy as jnp
from jax import lax
from jax.experimental import pallas as pl
from jax.experimental.pallas import tpu as pltpu
from jax.experimental.pallas import tpu_sc as plsc

N = 10000
E = 160000
D = 128
R = 8
BN = 1000
NB = N // BN

NTILES = 32
CH = 128
NCH = 40
ET = NCH * CH
EP = NTILES * ET
NP = 10240
TQ = N * R + 64
ROWS_PT = NP // NTILES
ROWS_SC = NP // 16
RCH = 64
ZB = 16


def _dense_body(x_ref, w_ref, xw_ref):
    xw_ref[...] = jnp.dot(x_ref[...], w_ref[0],
                          preferred_element_type=jnp.float32)


def _scal_body(x_ref, w_ref, q_ref, k_ref, gq_ref, gk_ref):
    wq = jnp.einsum('rdf,fo->rd', w_ref[...], q_ref[...],
                    preferred_element_type=jnp.float32)
    wk = jnp.einsum('rdf,fo->rd', w_ref[...], k_ref[...],
                    preferred_element_type=jnp.float32)
    xb = x_ref[...]
    gq_ref[...] = jnp.dot(xb, wq.T, preferred_element_type=jnp.float32)
    gk_ref[...] = jnp.dot(xb, wk.T, preferred_element_type=jnp.float32)


def _dense(x, weight, q, k_att):
    xw = pl.pallas_call(
        _dense_body,
        grid=(NB, R),
        in_specs=[
            pl.BlockSpec((BN, D), lambda i, r: (i, 0)),
            pl.BlockSpec((1, D, D), lambda i, r: (r, 0, 0)),
        ],
        out_specs=pl.BlockSpec((BN, D), lambda i, r: (r * NB + i, 0)),
        out_shape=jax.ShapeDtypeStruct((R * N, D), jnp.float32),
    )(x, weight)
    gq, gk = pl.pallas_call(
        _scal_body,
        grid=(NB,),
        in_specs=[
            pl.BlockSpec((BN, D), lambda i: (i, 0)),
            pl.BlockSpec((R, D, D), lambda i: (0, 0, 0)),
            pl.BlockSpec((D, 1), lambda i: (0, 0)),
            pl.BlockSpec((D, 1), lambda i: (0, 0)),
        ],
        out_specs=[
            pl.BlockSpec((BN, R), lambda i: (i, 0)),
            pl.BlockSpec((BN, R), lambda i: (i, 0)),
        ],
        out_shape=[
            jax.ShapeDtypeStruct((N, R), jnp.float32),
            jax.ShapeDtypeStruct((N, R), jnp.float32),
        ],
    )(x, weight, q, k_att)
    return xw, gq, gk


_MESH = plsc.VectorSubcoreMesh(core_axis_name="c", subcore_axis_name="s")


def _edge_kernel(src_h, dst_h, et_h, gq_h, gk_h, xw_h,
                 ex_h, dpart_h, apart_h,
                 dst_v, src_v, et_v, ex_v, ig, isc, zbuf, dz, g1, g2,
                 denom_sh, aggr_sh, sem):
    c = lax.axis_index("c")
    s = lax.axis_index("s")
    wid = c * 16 + s

    z16 = jnp.zeros((16,), jnp.float32)
    def _zrow(i, carry):
        for u in range(8):
            zbuf[i, pl.ds(u * 16, 16)] = z16
        return carry
    lax.fori_loop(0, ZB, _zrow, 0)
    def _zdz(i, carry):
        dz[pl.ds(i * 16, 16)] = z16
        return carry
    lax.fori_loop(0, ROWS_SC // 16, _zdz, 0)
    rbase = s * ROWS_SC
    def _zsp(i, carry):
        pltpu.sync_copy(zbuf, aggr_sh.at[pl.ds(rbase + i * ZB, ZB)])
        return carry
    lax.fori_loop(0, ROWS_SC // ZB, _zsp, 0)
    pltpu.sync_copy(dz, denom_sh.at[pl.ds(rbase, ROWS_SC)])
    plsc.subcore_barrier()

    pltpu.sync_copy(src_h.at[wid], src_v)
    pltpu.sync_copy(dst_h.at[wid], dst_v)
    pltpu.sync_copy(et_h.at[wid], et_v)

    def _chAB(j, carry):
        for u in range(8):
            sl = pl.ds(u * 16, 16)
            ig[sl] = dst_v[j, sl] * 8 + et_v[j, sl]
        pltpu.async_copy(gq_h.at[ig], g1, sem).wait()
        for u in range(8):
            sl = pl.ds(u * 16, 16)
            ig[sl] = src_v[j, sl] * 8 + et_v[j, sl]
        pltpu.async_copy(gk_h.at[ig], g2, sem).wait()
        for u in range(8):
            sl = pl.ds(u * 16, 16)
            pre = g1[sl] + g2[sl]
            alpha = jnp.where(pre >= 0, pre, 0.2 * pre)
            ex_v[j, sl] = jnp.exp(alpha)
        return carry
    lax.fori_loop(0, NCH, _chAB, 0)

    pltpu.sync_copy(ex_v, ex_h.at[wid])

    def _chD(j, carry):
        for u in range(8):
            sl = pl.ds(u * 16, 16)
            isc[sl] = dst_v[j, sl]
        pltpu.sync_copy(ex_v.at[j], denom_sh.at[isc], add=True)
        return carry
    lax.fori_loop(0, NCH, _chD, 0)

    def _msg_phase(rows):
        def _ch3(j, carry):
            for u in range(8):
                sl = pl.ds(u * 16, 16)
                ig[sl] = et_v[j, sl] * N + src_v[j, sl]
                isc[sl] = dst_v[j, sl]
            pltpu.async_copy(xw_h.at[ig], rows, sem).wait()

            def _e16(u, carry2):
                exv = ex_v[j, pl.ds(u * 16, 16)]
                for lane in range(16):
                    b = jnp.full((16,), exv[lane], jnp.float32)
                    er = u * 16 + lane
                    for h in range(8):
                        hs = pl.ds(h * 16, 16)
                        rows[er, hs] = rows[er, hs] * b
                return carry2
            lax.fori_loop(0, CH // 16, _e16, 0)
            pltpu.sync_copy(rows, aggr_sh.at[isc], add=True)
            return carry
        lax.fori_loop(0, NCH, _ch3, 0)

    pl.run_scoped(_msg_phase, pltpu.VMEM((CH, D), jnp.float32))

    plsc.subcore_barrier()
    pltpu.sync_copy(denom_sh.at[pl.ds(rbase, ROWS_SC)], dz)
    pltpu.sync_copy(dz, dpart_h.at[pl.ds(c * NP + rbase, ROWS_SC)])
    def _pub(i, carry):
        pltpu.sync_copy(aggr_sh.at[pl.ds(rbase + i * ZB, ZB)], zbuf)
        pltpu.sync_copy(zbuf, apart_h.at[c, pl.ds(rbase + i * ZB, ZB)])
        return carry
    lax.fori_loop(0, ROWS_SC // ZB, _pub, 0)


def _finalize_kernel(dst_h, ex_h, dpart_h, apart_h, x_h, bias_h,
                     att_h, out_h,
                     d0_v, d1_v, recip_v, dst_v, ex_v, att_v,
                     a0, a1, xb, bias_v):
    c = lax.axis_index("c")
    s = lax.axis_index("s")
    wid = c * 16 + s

    pltpu.sync_copy(dpart_h.at[pl.ds(0, NP)], d0_v)
    pltpu.sync_copy(dpart_h.at[pl.ds(NP, NP)], d1_v)
    pltpu.sync_copy(bias_h, bias_v)

    def _rc(i, carry):
        sl = pl.ds(i * 16, 16)
        recip_v[sl] = 1.0 / (d0_v[sl] + d1_v[sl] + 1e-16)
        return carry
    lax.fori_loop(0, NP // 16, _rc, 0)

    pltpu.sync_copy(dst_h.at[wid], dst_v)
    pltpu.sync_copy(ex_h.at[wid], ex_v)

    def _chA(j, carry):
        for u in range(8):
            sl = pl.ds(u * 16, 16)
            att_v[j, sl] = ex_v[j, sl] * plsc.load_gather(
                recip_v, [dst_v[j, sl]])
        return carry
    lax.fori_loop(0, NCH, _chA, 0)
    pltpu.sync_copy(att_v, att_h.at[wid])

    def _rowchunk(cc, carry):
        rb = wid * ROWS_PT + cc * RCH
        pltpu.sync_copy(apart_h.at[0, pl.ds(rb, RCH)], a0)
        pltpu.sync_copy(apart_h.at[1, pl.ds(rb, RCH)], a1)
        pltpu.sync_copy(x_h.at[pl.ds(rb, RCH)], xb)
        for g in range(RCH // 16):
            rv = recip_v[pl.ds(rb + g * 16, 16)]
            for lane in range(16):
                b = jnp.full((16,), rv[lane], jnp.float32)
                rr = g * 16 + lane
                for h in range(8):
                    hs = pl.ds(h * 16, 16)
                    a0[rr, hs] = ((a0[rr, hs] + a1[rr, hs]) * b
                                  + xb[rr, hs] + bias_v[hs])
        pltpu.sync_copy(a0, out_h.at[pl.ds(rb, RCH)])
        return carry
    lax.fori_loop(0, ROWS_PT // RCH, _rowchunk, 0)


@functools.partial(
    pl.kernel,
    mesh=_MESH,
    compiler_params=pltpu.CompilerParams(needs_layout_passes=False),
    out_type=[
        jax.ShapeDtypeStruct((NTILES, NCH, CH), jnp.float32),
        jax.ShapeDtypeStruct((2 * NP,), jnp.float32),
        jax.ShapeDtypeStruct((2, NP, D), jnp.float32),
    ],
    scratch_types=[
        pltpu.VMEM((NCH, CH), jnp.int32),
        pltpu.VMEM((NCH, CH), jnp.int32),
        pltpu.VMEM((NCH, CH), jnp.int32),
        pltpu.VMEM((NCH, CH), jnp.float32),
        pltpu.VMEM((CH,), jnp.int32),
        pltpu.VMEM((CH,), jnp.int32),
        pltpu.VMEM((ZB, D), jnp.float32),
        pltpu.VMEM((ROWS_SC,), jnp.float32),
        pltpu.VMEM((CH,), jnp.float32),
        pltpu.VMEM((CH,), jnp.float32),
        pltpu.VMEM_SHARED((NP,), jnp.float32),
        pltpu.VMEM_SHARED((NP, D), jnp.float32),
        pltpu.SemaphoreType.DMA,
    ],
)
def _edge(src_h, dst_h, et_h, gq_h, gk_h, xw_h, *rest):
    _edge_kernel(src_h, dst_h, et_h, gq_h, gk_h, xw_h, *rest)


@functools.partial(
    pl.kernel,
    mesh=_MESH,
    compiler_params=pltpu.CompilerParams(needs_layout_passes=False),
    out_type=[
        jax.ShapeDtypeStruct((NTILES, NCH, CH), jnp.float32),
        jax.ShapeDtypeStruct((NP, D), jnp.float32),
    ],
    scratch_types=[
        pltpu.VMEM((NP,), jnp.float32),
        pltpu.VMEM((NP,), jnp.float32),
        pltpu.VMEM((NP,), jnp.float32),
        pltpu.VMEM((NCH, CH), jnp.int32),
        pltpu.VMEM((NCH, CH), jnp.float32),
        pltpu.VMEM((NCH, CH), jnp.float32),
        pltpu.VMEM((RCH, D), jnp.float32),
        pltpu.VMEM((RCH, D), jnp.float32),
        pltpu.VMEM((RCH, D), jnp.float32),
        pltpu.VMEM((D,), jnp.float32),
    ],
)
def _finalize(dst_h, ex_h, dpart_h, apart_h, x_h, bias_h, *rest):
    _finalize_kernel(dst_h, ex_h, dpart_h, apart_h, x_h, bias_h, *rest)


@jax.jit
def _run(x, edge_index, edge_type, weight, q, k_att, bias):
    src = edge_index[0]
    dst = edge_index[1]
    et = edge_type

    xw_flat, gq, gk = _dense(x, weight, q, k_att)
    gq_pad = jnp.pad(gq.reshape(-1), (0, TQ - N * R))
    gk_pad = jnp.pad(gk.reshape(-1), (0, TQ - N * R))

    pad = EP - E
    src_p = jnp.concatenate([src, jnp.zeros((pad,), jnp.int32)]
                            ).reshape(NTILES, NCH, CH)
    dst_p = jnp.concatenate([dst, jnp.full((pad,), N, jnp.int32)]
                            ).reshape(NTILES, NCH, CH)
    et_p = jnp.concatenate([et, jnp.zeros((pad,), jnp.int32)]
                           ).reshape(NTILES, NCH, CH)
    x_pad = jnp.pad(x, ((0, NP - N), (0, 0)))

    ex3, dpart, apart = _edge(src_p, dst_p, et_p, gq_pad, gk_pad, xw_flat)
    att3, out_pad = _finalize(dst_p, ex3, dpart, apart, x_pad, bias)

    out = out_pad[:N]
    att = att3.reshape(-1)[:E, None]
    return out, edge_index, att


def kernel(x, edge_index, edge_type, weight, q, k_att, bias):
    return _run(x, edge_index, edge_type, weight, q, k_att, bias)


# speedup vs baseline: 10.9506x; 1.0611x over previous
import functools

import jax
import jax.numpy as jnp
from jax import lax
from jax.experimental import pallas as pl
from jax.experimental.pallas import tpu as pltpu
from jax.experimental.pallas import tpu_sc as plsc

N = 10000
E = 160000
D = 128
R = 8
BN = 1000
NB = N // BN

NTILES = 32
CH = 128
NCH = 40
ET = NCH * CH
EP = NTILES * ET
NP = 10240
TQ = N * R + 64
ROWS_PT = NP // NTILES
ROWS_SC = NP // 16
RCH = 64
ZB = 16


def _dense_body(x_ref, w_ref, xw_ref):
    xw_ref[...] = jnp.dot(x_ref[...], w_ref[0],
                          preferred_element_type=jnp.float32)


def _scal_body(x_ref, w_ref, q_ref, k_ref, gq_ref, gk_ref):
    wq = jnp.einsum('rdf,fo->rd', w_ref[...], q_ref[...],
                    preferred_element_type=jnp.float32)
    wk = jnp.einsum('rdf,fo->rd', w_ref[...], k_ref[...],
                    preferred_element_type=jnp.float32)
    xb = x_ref[...]
    gq_ref[...] = jnp.dot(xb, wq.T, preferred_element_type=jnp.float32)
    gk_ref[...] = jnp.dot(xb, wk.T, preferred_element_type=jnp.float32)


def _dense(x, weight, q, k_att):
    xw = pl.pallas_call(
        _dense_body,
        grid=(NB, R),
        in_specs=[
            pl.BlockSpec((BN, D), lambda i, r: (i, 0)),
            pl.BlockSpec((1, D, D), lambda i, r: (r, 0, 0)),
        ],
        out_specs=pl.BlockSpec((BN, D), lambda i, r: (r * NB + i, 0)),
        out_shape=jax.ShapeDtypeStruct((R * N, D), jnp.float32),
    )(x, weight)
    gq, gk = pl.pallas_call(
        _scal_body,
        grid=(NB,),
        in_specs=[
            pl.BlockSpec((BN, D), lambda i: (i, 0)),
            pl.BlockSpec((R, D, D), lambda i: (0, 0, 0)),
            pl.BlockSpec((D, 1), lambda i: (0, 0)),
            pl.BlockSpec((D, 1), lambda i: (0, 0)),
        ],
        out_specs=[
            pl.BlockSpec((BN, R), lambda i: (i, 0)),
            pl.BlockSpec((BN, R), lambda i: (i, 0)),
        ],
        out_shape=[
            jax.ShapeDtypeStruct((N, R), jnp.float32),
            jax.ShapeDtypeStruct((N, R), jnp.float32),
        ],
    )(x, weight, q, k_att)
    return xw, gq, gk


_MESH = plsc.VectorSubcoreMesh(core_axis_name="c", subcore_axis_name="s")


def _edge_kernel(src_h, dst_h, et_h, gq_h, gk_h, xw_h,
                 ex_h, dpart_h, apart_h,
                 dst_v, src_v, et_v, ex_v, ig, ik, zbuf, dz, g1, g2,
                 denom_sh, aggr_sh, sem_g, sem_k):
    c = lax.axis_index("c")
    s = lax.axis_index("s")
    wid = c * 16 + s

    z16 = jnp.zeros((16,), jnp.float32)
    def _zrow(i, carry):
        for u in range(8):
            zbuf[i, pl.ds(u * 16, 16)] = z16
        return carry
    lax.fori_loop(0, ZB, _zrow, 0)
    def _zdz(i, carry):
        dz[pl.ds(i * 16, 16)] = z16
        return carry
    lax.fori_loop(0, ROWS_SC // 16, _zdz, 0)
    rbase = s * ROWS_SC
    def _zsp(i, carry):
        pltpu.sync_copy(zbuf, aggr_sh.at[pl.ds(rbase + i * ZB, ZB)])
        return carry
    lax.fori_loop(0, ROWS_SC // ZB, _zsp, 0)
    pltpu.sync_copy(dz, denom_sh.at[pl.ds(rbase, ROWS_SC)])
    plsc.subcore_barrier()

    pltpu.sync_copy(src_h.at[wid], src_v)
    pltpu.sync_copy(dst_h.at[wid], dst_v)
    pltpu.sync_copy(et_h.at[wid], et_v)

    def _chAB(j, carry):
        for u in range(CH // 16):
            sl = pl.ds(u * 16, 16)
            ig[sl] = dst_v[j, sl] * 8 + et_v[j, sl]
            ik[sl] = src_v[j, sl] * 8 + et_v[j, sl]
        cp1 = pltpu.async_copy(gq_h.at[ig], g1, sem_g)
        cp2 = pltpu.async_copy(gk_h.at[ik], g2, sem_k)
        cp1.wait()
        cp2.wait()
        for u in range(CH // 16):
            sl = pl.ds(u * 16, 16)
            pre = g1[sl] + g2[sl]
            alpha = jnp.where(pre >= 0, pre, 0.2 * pre)
            ex_v[j, sl] = jnp.exp(alpha)
        return carry
    lax.fori_loop(0, NCH, _chAB, 0)

    pltpu.sync_copy(ex_v, ex_h.at[wid])

    def _chD(j, carry):
        pltpu.sync_copy(ex_v.at[j], denom_sh.at[dst_v.at[j]], add=True)
        return carry
    lax.fori_loop(0, NCH, _chD, 0)

    def _msg_phase(rows):
        def _ch3(j, carry):
            for u in range(CH // 16):
                sl = pl.ds(u * 16, 16)
                ig[sl] = et_v[j, sl] * N + src_v[j, sl]
            pltpu.async_copy(xw_h.at[ig], rows, sem_g).wait()

            def _e16(u, carry2):
                exv = ex_v[j, pl.ds(u * 16, 16)]
                for lane in range(16):
                    bc = jnp.full((16,), exv[lane], jnp.float32)
                    er = u * 16 + lane
                    for h in range(8):
                        hs = pl.ds(h * 16, 16)
                        rows[er, hs] = rows[er, hs] * bc
                return carry2
            lax.fori_loop(0, CH // 16, _e16, 0)
            pltpu.sync_copy(rows, aggr_sh.at[dst_v.at[j]], add=True)
            return carry
        lax.fori_loop(0, NCH, _ch3, 0)

    pl.run_scoped(_msg_phase, pltpu.VMEM((CH, D), jnp.float32))

    plsc.subcore_barrier()
    pltpu.sync_copy(denom_sh.at[pl.ds(rbase, ROWS_SC)], dz)
    pltpu.sync_copy(dz, dpart_h.at[pl.ds(c * NP + rbase, ROWS_SC)])
    def _pub(i, carry):
        pltpu.sync_copy(aggr_sh.at[pl.ds(rbase + i * ZB, ZB)], zbuf)
        pltpu.sync_copy(zbuf, apart_h.at[c, pl.ds(rbase + i * ZB, ZB)])
        return carry
    lax.fori_loop(0, ROWS_SC // ZB, _pub, 0)


def _finalize_kernel(dst_h, ex_h, dpart_h, apart_h, x_h, bias_h,
                     att_h, out_h,
                     d0_v, d1_v, recip_v, dst_v, ex_v, att_v,
                     a0, a1, xb, bias_v):
    c = lax.axis_index("c")
    s = lax.axis_index("s")
    wid = c * 16 + s

    pltpu.sync_copy(dpart_h.at[pl.ds(0, NP)], d0_v)
    pltpu.sync_copy(dpart_h.at[pl.ds(NP, NP)], d1_v)
    pltpu.sync_copy(bias_h, bias_v)

    def _rc(i, carry):
        sl = pl.ds(i * 16, 16)
        recip_v[sl] = 1.0 / (d0_v[sl] + d1_v[sl] + 1e-16)
        return carry
    lax.fori_loop(0, NP // 16, _rc, 0)

    pltpu.sync_copy(dst_h.at[wid], dst_v)
    pltpu.sync_copy(ex_h.at[wid], ex_v)

    def _chA(j, carry):
        for u in range(CH // 16):
            sl = pl.ds(u * 16, 16)
            att_v[j, sl] = ex_v[j, sl] * plsc.load_gather(
                recip_v, [dst_v[j, sl]])
        return carry
    lax.fori_loop(0, NCH, _chA, 0)
    pltpu.sync_copy(att_v, att_h.at[wid])

    def _rowchunk(cc, carry):
        rb = wid * ROWS_PT + cc * RCH
        pltpu.sync_copy(apart_h.at[0, pl.ds(rb, RCH)], a0)
        pltpu.sync_copy(apart_h.at[1, pl.ds(rb, RCH)], a1)
        pltpu.sync_copy(x_h.at[pl.ds(rb, RCH)], xb)
        for g in range(RCH // 16):
            rv = recip_v[pl.ds(rb + g * 16, 16)]
            for lane in range(16):
                b = jnp.full((16,), rv[lane], jnp.float32)
                rr = g * 16 + lane
                for h in range(8):
                    hs = pl.ds(h * 16, 16)
                    a0[rr, hs] = ((a0[rr, hs] + a1[rr, hs]) * b
                                  + xb[rr, hs] + bias_v[hs])
        pltpu.sync_copy(a0, out_h.at[pl.ds(rb, RCH)])
        return carry
    lax.fori_loop(0, ROWS_PT // RCH, _rowchunk, 0)


@functools.partial(
    pl.kernel,
    mesh=_MESH,
    compiler_params=pltpu.CompilerParams(needs_layout_passes=False),
    out_type=[
        jax.ShapeDtypeStruct((NTILES, NCH, CH), jnp.float32),
        jax.ShapeDtypeStruct((2 * NP,), jnp.float32),
        jax.ShapeDtypeStruct((2, NP, D), jnp.float32),
    ],
    scratch_types=[
        pltpu.VMEM((NCH, CH), jnp.int32),
        pltpu.VMEM((NCH, CH), jnp.int32),
        pltpu.VMEM((NCH, CH), jnp.int32),
        pltpu.VMEM((NCH, CH), jnp.float32),
        pltpu.VMEM((CH,), jnp.int32),
        pltpu.VMEM((CH,), jnp.int32),
        pltpu.VMEM((ZB, D), jnp.float32),
        pltpu.VMEM((ROWS_SC,), jnp.float32),
        pltpu.VMEM((CH,), jnp.float32),
        pltpu.VMEM((CH,), jnp.float32),
        pltpu.VMEM_SHARED((NP,), jnp.float32),
        pltpu.VMEM_SHARED((NP, D), jnp.float32),
        pltpu.SemaphoreType.DMA,
        pltpu.SemaphoreType.DMA,
    ],
)
def _edge(src_h, dst_h, et_h, gq_h, gk_h, xw_h, *rest):
    _edge_kernel(src_h, dst_h, et_h, gq_h, gk_h, xw_h, *rest)


@functools.partial(
    pl.kernel,
    mesh=_MESH,
    compiler_params=pltpu.CompilerParams(needs_layout_passes=False),
    out_type=[
        jax.ShapeDtypeStruct((NTILES, NCH, CH), jnp.float32),
        jax.ShapeDtypeStruct((NP, D), jnp.float32),
    ],
    scratch_types=[
        pltpu.VMEM((NP,), jnp.float32),
        pltpu.VMEM((NP,), jnp.float32),
        pltpu.VMEM((NP,), jnp.float32),
        pltpu.VMEM((NCH, CH), jnp.int32),
        pltpu.VMEM((NCH, CH), jnp.float32),
        pltpu.VMEM((NCH, CH), jnp.float32),
        pltpu.VMEM((RCH, D), jnp.float32),
        pltpu.VMEM((RCH, D), jnp.float32),
        pltpu.VMEM((RCH, D), jnp.float32),
        pltpu.VMEM((D,), jnp.float32),
    ],
)
def _finalize(dst_h, ex_h, dpart_h, apart_h, x_h, bias_h, *rest):
    _finalize_kernel(dst_h, ex_h, dpart_h, apart_h, x_h, bias_h, *rest)


@jax.jit
def _run(x, edge_index, edge_type, weight, q, k_att, bias):
    src = edge_index[0]
    dst = edge_index[1]
    et = edge_type

    xw_flat, gq, gk = _dense(x, weight, q, k_att)
    gq_pad = jnp.pad(gq.reshape(-1), (0, TQ - N * R))
    gk_pad = jnp.pad(gk.reshape(-1), (0, TQ - N * R))

    pad = EP - E
    src_p = jnp.concatenate([src, jnp.zeros((pad,), jnp.int32)]
                            ).reshape(NTILES, NCH, CH)
    dst_p = jnp.concatenate([dst, jnp.full((pad,), N, jnp.int32)]
                            ).reshape(NTILES, NCH, CH)
    et_p = jnp.concatenate([et, jnp.zeros((pad,), jnp.int32)]
                           ).reshape(NTILES, NCH, CH)
    x_pad = jnp.pad(x, ((0, NP - N), (0, 0)))

    ex3, dpart, apart = _edge(src_p, dst_p, et_p, gq_pad, gk_pad, xw_flat)
    att3, out_pad = _finalize(dst_p, ex3, dpart, apart, x_pad, bias)

    out = out_pad[:N]
    att = att3.reshape(-1)[:E, None]
    return out, edge_index, att


def kernel(x, edge_index, edge_type, weight, q, k_att, bias):
    return _run(x, edge_index, edge_type, weight, q, k_att, bias)


# speedup vs baseline: 12.5390x; 1.1451x over previous
import functools

import jax
import jax.numpy as jnp
from jax import lax
from jax.experimental import pallas as pl
from jax.experimental.pallas import tpu as pltpu
from jax.experimental.pallas import tpu_sc as plsc

N = 10000
E = 160000
D = 128
R = 8
BN = 1000
NB = N // BN

NTILES = 32
CH = 80
NCH = 64
ET = NCH * CH
EP = NTILES * ET
NP = 10240
TQ = N * R + 64
ROWS_PT = NP // NTILES
ROWS_SC = NP // 16
RCH = 64
ZB = 16
NCHH = 32


def _dense_body(x_ref, w_ref, xw_ref):
    xw_ref[...] = jnp.dot(x_ref[...], w_ref[0],
                          preferred_element_type=jnp.float32)


def _scal_body(x_ref, w_ref, q_ref, k_ref, gq_ref, gk_ref):
    wq = jnp.einsum('rdf,fo->rd', w_ref[...], q_ref[...],
                    preferred_element_type=jnp.float32)
    wk = jnp.einsum('rdf,fo->rd', w_ref[...], k_ref[...],
                    preferred_element_type=jnp.float32)
    xb = x_ref[...]
    gq_ref[...] = jnp.dot(xb, wq.T, preferred_element_type=jnp.float32)
    gk_ref[...] = jnp.dot(xb, wk.T, preferred_element_type=jnp.float32)


def _dense(x, weight, q, k_att):
    xw = pl.pallas_call(
        _dense_body,
        grid=(NB, R),
        in_specs=[
            pl.BlockSpec((BN, D), lambda i, r: (i, 0)),
            pl.BlockSpec((1, D, D), lambda i, r: (r, 0, 0)),
        ],
        out_specs=pl.BlockSpec((BN, D), lambda i, r: (r * NB + i, 0)),
        out_shape=jax.ShapeDtypeStruct((R * N, D), jnp.float32),
    )(x, weight)
    gq, gk = pl.pallas_call(
        _scal_body,
        grid=(NB,),
        in_specs=[
            pl.BlockSpec((BN, D), lambda i: (i, 0)),
            pl.BlockSpec((R, D, D), lambda i: (0, 0, 0)),
            pl.BlockSpec((D, 1), lambda i: (0, 0)),
            pl.BlockSpec((D, 1), lambda i: (0, 0)),
        ],
        out_specs=[
            pl.BlockSpec((BN, R), lambda i: (i, 0)),
            pl.BlockSpec((BN, R), lambda i: (i, 0)),
        ],
        out_shape=[
            jax.ShapeDtypeStruct((N, R), jnp.float32),
            jax.ShapeDtypeStruct((N, R), jnp.float32),
        ],
    )(x, weight, q, k_att)
    return xw, gq, gk


_MESH = plsc.VectorSubcoreMesh(core_axis_name="c", subcore_axis_name="s")


def _edge_kernel(src_h, dst_h, et_h, gq_h, gk_h, xw_h,
                 ex_h, dpart_h, apart_h,
                 dst_v, src_v, et_v, ex_v, ig, ik, zbuf, dz, g1, g2,
                 denom_sh, aggr_sh, sem_g, sem_k):
    c = lax.axis_index("c")
    s = lax.axis_index("s")
    wid = c * 16 + s

    z16 = jnp.zeros((16,), jnp.float32)
    def _zrow(i, carry):
        for u in range(8):
            zbuf[i, pl.ds(u * 16, 16)] = z16
        return carry
    lax.fori_loop(0, ZB, _zrow, 0)
    def _zdz(i, carry):
        dz[pl.ds(i * 16, 16)] = z16
        return carry
    lax.fori_loop(0, ROWS_SC // 16, _zdz, 0)
    rbase = s * ROWS_SC
    def _zsp(i, carry):
        pltpu.sync_copy(zbuf, aggr_sh.at[pl.ds(rbase + i * ZB, ZB)])
        return carry
    lax.fori_loop(0, ROWS_SC // ZB, _zsp, 0)
    pltpu.sync_copy(dz, denom_sh.at[pl.ds(rbase, ROWS_SC)])
    plsc.subcore_barrier()

    for half in range(2):
        jo = half * NCHH

        pltpu.sync_copy(src_h.at[wid, pl.ds(jo, NCHH)], src_v)
        pltpu.sync_copy(dst_h.at[wid, pl.ds(jo, NCHH)], dst_v)
        pltpu.sync_copy(et_h.at[wid, pl.ds(jo, NCHH)], et_v)

        def _chAB(j, carry):
            for u in range(CH // 16):
                sl = pl.ds(u * 16, 16)
                ig[sl] = dst_v[j, sl] * 8 + et_v[j, sl]
                ik[sl] = src_v[j, sl] * 8 + et_v[j, sl]
            cp1 = pltpu.async_copy(gq_h.at[ig], g1, sem_g)
            cp2 = pltpu.async_copy(gk_h.at[ik], g2, sem_k)
            cp1.wait()
            cp2.wait()
            for u in range(CH // 16):
                sl = pl.ds(u * 16, 16)
                pre = g1[sl] + g2[sl]
                alpha = jnp.where(pre >= 0, pre, 0.2 * pre)
                ex_v[j, sl] = jnp.exp(alpha)
            return carry
        lax.fori_loop(0, NCHH, _chAB, 0)

        pltpu.sync_copy(ex_v, ex_h.at[wid, pl.ds(jo, NCHH)])

        def _chD(j, carry):
            pltpu.sync_copy(ex_v.at[j], denom_sh.at[dst_v.at[j]], add=True)
            return carry
        lax.fori_loop(0, NCHH, _chD, 0)

        def _ridx(j, carry):
            for u in range(CH // 16):
                sl = pl.ds(u * 16, 16)
                src_v[j, sl] = et_v[j, sl] * N + src_v[j, sl]
            return carry
        lax.fori_loop(0, NCHH, _ridx, 0)

        def _msg_phase(r0, r1):
            rows = (r0, r1)
            gsems = (sem_g, sem_k)
            pltpu.async_copy(xw_h.at[src_v.at[0]], rows[0], gsems[0])

            def _step(j, b):
                pltpu.make_async_copy(xw_h.at[src_v.at[j]], rows[b],
                                      gsems[b]).wait()

                @pl.when(j + 1 < NCHH)
                def _():
                    pltpu.async_copy(xw_h.at[src_v.at[j + 1]], rows[1 - b],
                                     gsems[1 - b])

                def _e16(u, carry2):
                    exv = ex_v[j, pl.ds(u * 16, 16)]
                    for lane in range(16):
                        bc = jnp.full((16,), exv[lane], jnp.float32)
                        er = u * 16 + lane
                        for h in range(8):
                            hs = pl.ds(h * 16, 16)
                            rows[b][er, hs] = rows[b][er, hs] * bc
                    return carry2
                lax.fori_loop(0, CH // 16, _e16, 0)
                pltpu.sync_copy(rows[b], aggr_sh.at[dst_v.at[j]], add=True)

            def _ring(i, carry):
                for b in range(2):
                    _step(i * 2 + b, b)
                return carry
            lax.fori_loop(0, NCHH // 2, _ring, 0)

        pl.run_scoped(_msg_phase, pltpu.VMEM((CH, D), jnp.float32),
                      pltpu.VMEM((CH, D), jnp.float32))

    plsc.subcore_barrier()
    pltpu.sync_copy(denom_sh.at[pl.ds(rbase, ROWS_SC)], dz)
    pltpu.sync_copy(dz, dpart_h.at[pl.ds(c * NP + rbase, ROWS_SC)])
    def _pub(i, carry):
        pltpu.sync_copy(aggr_sh.at[pl.ds(rbase + i * ZB, ZB)], zbuf)
        pltpu.sync_copy(zbuf, apart_h.at[c, pl.ds(rbase + i * ZB, ZB)])
        return carry
    lax.fori_loop(0, ROWS_SC // ZB, _pub, 0)


def _finalize_kernel(dst_h, ex_h, dpart_h, apart_h, x_h, bias_h,
                     att_h, out_h,
                     d0_v, d1_v, recip_v, dst_v, ex_v, att_v,
                     a0, a1, xb, bias_v):
    c = lax.axis_index("c")
    s = lax.axis_index("s")
    wid = c * 16 + s

    pltpu.sync_copy(dpart_h.at[pl.ds(0, NP)], d0_v)
    pltpu.sync_copy(dpart_h.at[pl.ds(NP, NP)], d1_v)
    pltpu.sync_copy(bias_h, bias_v)

    def _rc(i, carry):
        sl = pl.ds(i * 16, 16)
        recip_v[sl] = 1.0 / (d0_v[sl] + d1_v[sl] + 1e-16)
        return carry
    lax.fori_loop(0, NP // 16, _rc, 0)

    pltpu.sync_copy(dst_h.at[wid], dst_v)
    pltpu.sync_copy(ex_h.at[wid], ex_v)

    def _chA(j, carry):
        for u in range(CH // 16):
            sl = pl.ds(u * 16, 16)
            att_v[j, sl] = ex_v[j, sl] * plsc.load_gather(
                recip_v, [dst_v[j, sl]])
        return carry
    lax.fori_loop(0, NCH, _chA, 0)
    pltpu.sync_copy(att_v, att_h.at[wid])

    def _rowchunk(cc, carry):
        rb = wid * ROWS_PT + cc * RCH
        pltpu.sync_copy(apart_h.at[0, pl.ds(rb, RCH)], a0)
        pltpu.sync_copy(apart_h.at[1, pl.ds(rb, RCH)], a1)
        pltpu.sync_copy(x_h.at[pl.ds(rb, RCH)], xb)
        for g in range(RCH // 16):
            rv = recip_v[pl.ds(rb + g * 16, 16)]
            for lane in range(16):
                b = jnp.full((16,), rv[lane], jnp.float32)
                rr = g * 16 + lane
                for h in range(8):
                    hs = pl.ds(h * 16, 16)
                    a0[rr, hs] = ((a0[rr, hs] + a1[rr, hs]) * b
                                  + xb[rr, hs] + bias_v[hs])
        pltpu.sync_copy(a0, out_h.at[pl.ds(rb, RCH)])
        return carry
    lax.fori_loop(0, ROWS_PT // RCH, _rowchunk, 0)


@functools.partial(
    pl.kernel,
    mesh=_MESH,
    compiler_params=pltpu.CompilerParams(needs_layout_passes=False),
    out_type=[
        jax.ShapeDtypeStruct((NTILES, NCH, CH), jnp.float32),
        jax.ShapeDtypeStruct((2 * NP,), jnp.float32),
        jax.ShapeDtypeStruct((2, NP, D), jnp.float32),
    ],
    scratch_types=[
        pltpu.VMEM((NCHH, CH), jnp.int32),
        pltpu.VMEM((NCHH, CH), jnp.int32),
        pltpu.VMEM((NCHH, CH), jnp.int32),
        pltpu.VMEM((NCHH, CH), jnp.float32),
        pltpu.VMEM((CH,), jnp.int32),
        pltpu.VMEM((CH,), jnp.int32),
        pltpu.VMEM((ZB, D), jnp.float32),
        pltpu.VMEM((ROWS_SC,), jnp.float32),
        pltpu.VMEM((CH,), jnp.float32),
        pltpu.VMEM((CH,), jnp.float32),
        pltpu.VMEM_SHARED((NP,), jnp.float32),
        pltpu.VMEM_SHARED((NP, D), jnp.float32),
        pltpu.SemaphoreType.DMA,
        pltpu.SemaphoreType.DMA,
    ],
)
def _edge(src_h, dst_h, et_h, gq_h, gk_h, xw_h, *rest):
    _edge_kernel(src_h, dst_h, et_h, gq_h, gk_h, xw_h, *rest)


@functools.partial(
    pl.kernel,
    mesh=_MESH,
    compiler_params=pltpu.CompilerParams(needs_layout_passes=False),
    out_type=[
        jax.ShapeDtypeStruct((NTILES, NCH, CH), jnp.float32),
        jax.ShapeDtypeStruct((NP, D), jnp.float32),
    ],
    scratch_types=[
        pltpu.VMEM((NP,), jnp.float32),
        pltpu.VMEM((NP,), jnp.float32),
        pltpu.VMEM((NP,), jnp.float32),
        pltpu.VMEM((NCH, CH), jnp.int32),
        pltpu.VMEM((NCH, CH), jnp.float32),
        pltpu.VMEM((NCH, CH), jnp.float32),
        pltpu.VMEM((RCH, D), jnp.float32),
        pltpu.VMEM((RCH, D), jnp.float32),
        pltpu.VMEM((RCH, D), jnp.float32),
        pltpu.VMEM((D,), jnp.float32),
    ],
)
def _finalize(dst_h, ex_h, dpart_h, apart_h, x_h, bias_h, *rest):
    _finalize_kernel(dst_h, ex_h, dpart_h, apart_h, x_h, bias_h, *rest)


@jax.jit
def _run(x, edge_index, edge_type, weight, q, k_att, bias):
    src = edge_index[0]
    dst = edge_index[1]
    et = edge_type

    xw_flat, gq, gk = _dense(x, weight, q, k_att)
    gq_pad = jnp.pad(gq.reshape(-1), (0, TQ - N * R))
    gk_pad = jnp.pad(gk.reshape(-1), (0, TQ - N * R))

    pad = EP - E
    src_p = jnp.concatenate([src, jnp.zeros((pad,), jnp.int32)]
                            ).reshape(NTILES, NCH, CH)
    dst_p = jnp.concatenate([dst, jnp.full((pad,), N, jnp.int32)]
                            ).reshape(NTILES, NCH, CH)
    et_p = jnp.concatenate([et, jnp.zeros((pad,), jnp.int32)]
                           ).reshape(NTILES, NCH, CH)
    x_pad = jnp.pad(x, ((0, NP - N), (0, 0)))

    ex3, dpart, apart = _edge(src_p, dst_p, et_p, gq_pad, gk_pad, xw_flat)
    att3, out_pad = _finalize(dst_p, ex3, dpart, apart, x_pad, bias)

    out = out_pad[:N]
    att = att3.reshape(-1)[:E, None]
    return out, edge_index, att


def kernel(x, edge_index, edge_type, weight, q, k_att, bias):
    return _run(x, edge_index, edge_type, weight, q, k_att, bias)


# speedup vs baseline: 12.5486x; 1.0008x over previous
import functools

import jax
import jax.numpy as jnp
from jax import lax
from jax.experimental import pallas as pl
from jax.experimental.pallas import tpu as pltpu
from jax.experimental.pallas import tpu_sc as plsc

N = 10000
E = 160000
D = 128
R = 8
BN = 1000
NB = N // BN

NTILES = 32
CH = 80
NCH = 64
ET = NCH * CH
EP = NTILES * ET
NP = 10240
TQ = N * R + 64
ROWS_PT = NP // NTILES
ROWS_SC = NP // 16
RCH = 64
ZB = 16
NCHH = 32


def _dense_body(x_ref, w_ref, xw_ref):
    xw_ref[...] = jnp.dot(x_ref[...].astype(jnp.bfloat16),
                          w_ref[0].astype(jnp.bfloat16),
                          preferred_element_type=jnp.float32)


def _scal_body(x_ref, w_ref, q_ref, k_ref, gq_ref, gk_ref):
    wq = jnp.einsum('rdf,fo->rd', w_ref[...], q_ref[...],
                    preferred_element_type=jnp.float32)
    wk = jnp.einsum('rdf,fo->rd', w_ref[...], k_ref[...],
                    preferred_element_type=jnp.float32)
    xb = x_ref[...]
    gq_ref[...] = jnp.dot(xb, wq.T, preferred_element_type=jnp.float32)
    gk_ref[...] = jnp.dot(xb, wk.T, preferred_element_type=jnp.float32)


def _dense(x, weight, q, k_att):
    xw = pl.pallas_call(
        _dense_body,
        grid=(NB, R),
        in_specs=[
            pl.BlockSpec((BN, D), lambda i, r: (i, 0)),
            pl.BlockSpec((1, D, D), lambda i, r: (r, 0, 0)),
        ],
        out_specs=pl.BlockSpec((BN, D), lambda i, r: (r * NB + i, 0)),
        out_shape=jax.ShapeDtypeStruct((R * N, D), jnp.float32),
    )(x, weight)
    gq, gk = pl.pallas_call(
        _scal_body,
        grid=(NB,),
        in_specs=[
            pl.BlockSpec((BN, D), lambda i: (i, 0)),
            pl.BlockSpec((R, D, D), lambda i: (0, 0, 0)),
            pl.BlockSpec((D, 1), lambda i: (0, 0)),
            pl.BlockSpec((D, 1), lambda i: (0, 0)),
        ],
        out_specs=[
            pl.BlockSpec((BN, R), lambda i: (i, 0)),
            pl.BlockSpec((BN, R), lambda i: (i, 0)),
        ],
        out_shape=[
            jax.ShapeDtypeStruct((N, R), jnp.float32),
            jax.ShapeDtypeStruct((N, R), jnp.float32),
        ],
    )(x, weight, q, k_att)
    return xw, gq, gk


_MESH = plsc.VectorSubcoreMesh(core_axis_name="c", subcore_axis_name="s")


def _edge_kernel(src_h, dst_h, et_h, gq_h, gk_h, xw_h,
                 ex_h, dpart_h, apart_h,
                 dst_v, src_v, et_v, ex_v, ig, ik, zbuf, dz, g1, g2,
                 denom_sh, aggr_sh, sem_g, sem_k):
    c = lax.axis_index("c")
    s = lax.axis_index("s")
    wid = c * 16 + s

    z16 = jnp.zeros((16,), jnp.float32)
    def _zrow(i, carry):
        for u in range(8):
            zbuf[i, pl.ds(u * 16, 16)] = z16
        return carry
    lax.fori_loop(0, ZB, _zrow, 0)
    def _zdz(i, carry):
        dz[pl.ds(i * 16, 16)] = z16
        return carry
    lax.fori_loop(0, ROWS_SC // 16, _zdz, 0)
    rbase = s * ROWS_SC
    def _zsp(i, carry):
        pltpu.sync_copy(zbuf, aggr_sh.at[pl.ds(rbase + i * ZB, ZB)])
        return carry
    lax.fori_loop(0, ROWS_SC // ZB, _zsp, 0)
    pltpu.sync_copy(dz, denom_sh.at[pl.ds(rbase, ROWS_SC)])
    plsc.subcore_barrier()

    for half in range(2):
        jo = half * NCHH

        pltpu.sync_copy(src_h.at[wid, pl.ds(jo, NCHH)], src_v)
        pltpu.sync_copy(dst_h.at[wid, pl.ds(jo, NCHH)], dst_v)
        pltpu.sync_copy(et_h.at[wid, pl.ds(jo, NCHH)], et_v)

        def _chAB(j, carry):
            for u in range(CH // 16):
                sl = pl.ds(u * 16, 16)
                ig[sl] = dst_v[j, sl] * 8 + et_v[j, sl]
                ik[sl] = src_v[j, sl] * 8 + et_v[j, sl]
            cp1 = pltpu.async_copy(gq_h.at[ig], g1, sem_g)
            cp2 = pltpu.async_copy(gk_h.at[ik], g2, sem_k)
            cp1.wait()
            cp2.wait()
            for u in range(CH // 16):
                sl = pl.ds(u * 16, 16)
                pre = g1[sl] + g2[sl]
                alpha = jnp.where(pre >= 0, pre, 0.2 * pre)
                ex_v[j, sl] = jnp.exp(alpha)
            return carry
        lax.fori_loop(0, NCHH, _chAB, 0)

        pltpu.sync_copy(ex_v, ex_h.at[wid, pl.ds(jo, NCHH)])

        def _chD(j, carry):
            pltpu.sync_copy(ex_v.at[j], denom_sh.at[dst_v.at[j]], add=True)
            return carry
        lax.fori_loop(0, NCHH, _chD, 0)

        def _ridx(j, carry):
            for u in range(CH // 16):
                sl = pl.ds(u * 16, 16)
                src_v[j, sl] = et_v[j, sl] * N + src_v[j, sl]
            return carry
        lax.fori_loop(0, NCHH, _ridx, 0)

        def _msg_phase(r0, r1):
            rows = (r0, r1)
            gsems = (sem_g, sem_k)
            pltpu.async_copy(xw_h.at[src_v.at[0]], rows[0], gsems[0])

            def _step(j, b):
                pltpu.make_async_copy(xw_h.at[src_v.at[j]], rows[b],
                                      gsems[b]).wait()

                @pl.when(j + 1 < NCHH)
                def _():
                    pltpu.async_copy(xw_h.at[src_v.at[j + 1]], rows[1 - b],
                                     gsems[1 - b])

                def _e16(u, carry2):
                    exv = ex_v[j, pl.ds(u * 16, 16)]
                    for lane in range(16):
                        bc = jnp.full((16,), exv[lane], jnp.float32)
                        er = u * 16 + lane
                        for h in range(8):
                            hs = pl.ds(h * 16, 16)
                            rows[b][er, hs] = rows[b][er, hs] * bc
                    return carry2
                lax.fori_loop(0, CH // 16, _e16, 0)
                pltpu.sync_copy(rows[b], aggr_sh.at[dst_v.at[j]], add=True)

            def _ring(i, carry):
                for b in range(2):
                    _step(i * 2 + b, b)
                return carry
            lax.fori_loop(0, NCHH // 2, _ring, 0)

        pl.run_scoped(_msg_phase, pltpu.VMEM((CH, D), jnp.float32),
                      pltpu.VMEM((CH, D), jnp.float32))

    plsc.subcore_barrier()
    pltpu.sync_copy(denom_sh.at[pl.ds(rbase, ROWS_SC)], dz)
    pltpu.sync_copy(dz, dpart_h.at[pl.ds(c * NP + rbase, ROWS_SC)])
    def _pub(i, carry):
        pltpu.sync_copy(aggr_sh.at[pl.ds(rbase + i * ZB, ZB)], zbuf)
        pltpu.sync_copy(zbuf, apart_h.at[c, pl.ds(rbase + i * ZB, ZB)])
        return carry
    lax.fori_loop(0, ROWS_SC // ZB, _pub, 0)


def _finalize_kernel(dst_h, ex_h, dpart_h, apart_h, x_h, bias_h,
                     att_h, out_h,
                     d0_v, d1_v, recip_v, dst_v, ex_v, att_v,
                     a0, a1, xb, bias_v):
    c = lax.axis_index("c")
    s = lax.axis_index("s")
    wid = c * 16 + s

    pltpu.sync_copy(dpart_h.at[pl.ds(0, NP)], d0_v)
    pltpu.sync_copy(dpart_h.at[pl.ds(NP, NP)], d1_v)
    pltpu.sync_copy(bias_h, bias_v)

    def _rc(i, carry):
        sl = pl.ds(i * 16, 16)
        recip_v[sl] = 1.0 / (d0_v[sl] + d1_v[sl] + 1e-16)
        return carry
    lax.fori_loop(0, NP // 16, _rc, 0)

    pltpu.sync_copy(dst_h.at[wid], dst_v)
    pltpu.sync_copy(ex_h.at[wid], ex_v)

    def _chA(j, carry):
        for u in range(CH // 16):
            sl = pl.ds(u * 16, 16)
            att_v[j, sl] = ex_v[j, sl] * plsc.load_gather(
                recip_v, [dst_v[j, sl]])
        return carry
    lax.fori_loop(0, NCH, _chA, 0)
    pltpu.sync_copy(att_v, att_h.at[wid])

    def _rowchunk(cc, carry):
        rb = wid * ROWS_PT + cc * RCH
        pltpu.sync_copy(apart_h.at[0, pl.ds(rb, RCH)], a0)
        pltpu.sync_copy(apart_h.at[1, pl.ds(rb, RCH)], a1)
        pltpu.sync_copy(x_h.at[pl.ds(rb, RCH)], xb)
        for g in range(RCH // 16):
            rv = recip_v[pl.ds(rb + g * 16, 16)]
            for lane in range(16):
                b = jnp.full((16,), rv[lane], jnp.float32)
                rr = g * 16 + lane
                for h in range(8):
                    hs = pl.ds(h * 16, 16)
                    a0[rr, hs] = ((a0[rr, hs] + a1[rr, hs]) * b
                                  + xb[rr, hs] + bias_v[hs])
        pltpu.sync_copy(a0, out_h.at[pl.ds(rb, RCH)])
        return carry
    lax.fori_loop(0, ROWS_PT // RCH, _rowchunk, 0)


@functools.partial(
    pl.kernel,
    mesh=_MESH,
    compiler_params=pltpu.CompilerParams(needs_layout_passes=False),
    out_type=[
        jax.ShapeDtypeStruct((NTILES, NCH, CH), jnp.float32),
        jax.ShapeDtypeStruct((2 * NP,), jnp.float32),
        jax.ShapeDtypeStruct((2, NP, D), jnp.float32),
    ],
    scratch_types=[
        pltpu.VMEM((NCHH, CH), jnp.int32),
        pltpu.VMEM((NCHH, CH), jnp.int32),
        pltpu.VMEM((NCHH, CH), jnp.int32),
        pltpu.VMEM((NCHH, CH), jnp.float32),
        pltpu.VMEM((CH,), jnp.int32),
        pltpu.VMEM((CH,), jnp.int32),
        pltpu.VMEM((ZB, D), jnp.float32),
        pltpu.VMEM((ROWS_SC,), jnp.float32),
        pltpu.VMEM((CH,), jnp.float32),
        pltpu.VMEM((CH,), jnp.float32),
        pltpu.VMEM_SHARED((NP,), jnp.float32),
        pltpu.VMEM_SHARED((NP, D), jnp.float32),
        pltpu.SemaphoreType.DMA,
        pltpu.SemaphoreType.DMA,
    ],
)
def _edge(src_h, dst_h, et_h, gq_h, gk_h, xw_h, *rest):
    _edge_kernel(src_h, dst_h, et_h, gq_h, gk_h, xw_h, *rest)


@functools.partial(
    pl.kernel,
    mesh=_MESH,
    compiler_params=pltpu.CompilerParams(needs_layout_passes=False),
    out_type=[
        jax.ShapeDtypeStruct((NTILES, NCH, CH), jnp.float32),
        jax.ShapeDtypeStruct((NP, D), jnp.float32),
    ],
    scratch_types=[
        pltpu.VMEM((NP,), jnp.float32),
        pltpu.VMEM((NP,), jnp.float32),
        pltpu.VMEM((NP,), jnp.float32),
        pltpu.VMEM((NCH, CH), jnp.int32),
        pltpu.VMEM((NCH, CH), jnp.float32),
        pltpu.VMEM((NCH, CH), jnp.float32),
        pltpu.VMEM((RCH, D), jnp.float32),
        pltpu.VMEM((RCH, D), jnp.float32),
        pltpu.VMEM((RCH, D), jnp.float32),
        pltpu.VMEM((D,), jnp.float32),
    ],
)
def _finalize(dst_h, ex_h, dpart_h, apart_h, x_h, bias_h, *rest):
    _finalize_kernel(dst_h, ex_h, dpart_h, apart_h, x_h, bias_h, *rest)


@jax.jit
def _run(x, edge_index, edge_type, weight, q, k_att, bias):
    src = edge_index[0]
    dst = edge_index[1]
    et = edge_type

    xw_flat, gq, gk = _dense(x, weight, q, k_att)
    gq_pad = jnp.pad(gq.reshape(-1), (0, TQ - N * R))
    gk_pad = jnp.pad(gk.reshape(-1), (0, TQ - N * R))

    pad = EP - E
    src_p = jnp.concatenate([src, jnp.zeros((pad,), jnp.int32)]
                            ).reshape(NTILES, NCH, CH)
    dst_p = jnp.concatenate([dst, jnp.full((pad,), N, jnp.int32)]
                            ).reshape(NTILES, NCH, CH)
    et_p = jnp.concatenate([et, jnp.zeros((pad,), jnp.int32)]
                           ).reshape(NTILES, NCH, CH)
    x_pad = jnp.pad(x, ((0, NP - N), (0, 0)))

    ex3, dpart, apart = _edge(src_p, dst_p, et_p, gq_pad, gk_pad, xw_flat)
    att3, out_pad = _finalize(dst_p, ex3, dpart, apart, x_pad, bias)

    out = out_pad[:N]
    att = att3.reshape(-1)[:E, None]
    return out, edge_index, att


def kernel(x, edge_index, edge_type, weight, q, k_att, bias):
    return _run(x, edge_index, edge_type, weight, q, k_att, bias)
